# Initial kernel scaffold; baseline (speedup 1.0000x reference)
import jax
import jax.numpy as jnp
from jax.experimental import pallas as pl


def kernel(x, edge_index, edge_attr, W1, b1, W2, b2, W3, b3, Wl, bl):
    raise NotImplementedError("write your pallas kernel here")


# speedup vs baseline: 19.9953x; 19.9953x over previous
import functools

import jax
import jax.numpy as jnp
from jax import lax
from jax.experimental import pallas as pl
from jax.experimental.pallas import tpu as pltpu
from jax.experimental.pallas import tpu_sc as plsc

N = 10000
E = 320000
N_PAD = 10240
NC, NS, L = 2, 16, 16
NW = NC * NS
CHUNK = 128
NCH = 80
EPW = NCH * CHUNK
E_PAD = NW * EPW
ZR = N_PAD // NS

_i32 = jnp.int32
_f32 = jnp.float32


def _full(v):
    return jnp.full((L,), v, dtype=_i32)


def _make_deg_kernel():
    mesh = plsc.VectorSubcoreMesh(core_axis_name="c", subcore_axis_name="s")

    @functools.partial(
        pl.kernel,
        out_type=jax.ShapeDtypeStruct((NC * N_PAD,), _f32),
        mesh=mesh,
        compiler_params=pltpu.CompilerParams(use_tc_tiling_on_sc=False),
        scratch_types=[
            pltpu.VMEM_SHARED((N_PAD,), _f32),
            pltpu.VMEM((NCH, CHUNK), _i32),
            pltpu.VMEM((EPW,), _f32),
            pltpu.VMEM((ZR,), _f32),
        ],
    )
    def deg_kernel(col_hbm, w_hbm, out_hbm, acc, col_v, w_v, zbuf):
        cid = lax.axis_index("c")
        sid = lax.axis_index("s")
        wid = cid * NS + sid
        for i in range(ZR // L):
            zbuf[pl.ds(i * L, L)] = jnp.zeros((L,), _f32)
        pltpu.sync_copy(zbuf, acc.at[pl.ds(sid * ZR, ZR)])
        pltpu.sync_copy(col_hbm.at[wid], col_v)
        pltpu.sync_copy(w_hbm.at[wid], w_v)
        plsc.subcore_barrier()

        @pl.loop(0, NCH)
        def _(j):
            pltpu.sync_copy(w_v.at[pl.ds(j * CHUNK, CHUNK)],
                            acc.at[col_v.at[j]], add=True)

        plsc.subcore_barrier()
        pltpu.sync_copy(acc.at[pl.ds(sid * ZR, ZR)],
                        out_hbm.at[pl.ds(cid * N_PAD + sid * ZR, ZR)])

    return deg_kernel


def _make_msg_kernel(C):
    mesh = plsc.VectorSubcoreMesh(core_axis_name="c", subcore_axis_name="s")

    @functools.partial(
        pl.kernel,
        out_type=jax.ShapeDtypeStruct((NC * N_PAD, C), _f32),
        mesh=mesh,
        compiler_params=pltpu.CompilerParams(
            needs_layout_passes=False, use_tc_tiling_on_sc=False),
        scratch_types=[
            pltpu.VMEM_SHARED((N_PAD, C), _f32),
            pltpu.VMEM((NCH, CHUNK), _i32),
            pltpu.VMEM((NCH, CHUNK), _i32),
            pltpu.VMEM((EPW,), _f32),
            pltpu.VMEM((CHUNK, C), _f32),
            pltpu.VMEM((CHUNK, C), _f32),
            pltpu.VMEM((32, C), _f32),
            pltpu.SemaphoreType.DMA,
            pltpu.SemaphoreType.DMA,
        ],
    )
    def msg_kernel(g_hbm, row_hbm, col_hbm, w_hbm, out_hbm,
                   acc, row_v, col_v, w_v, rows0, rows1, zbuf, sem0, sem1):
        cid = lax.axis_index("c")
        sid = lax.axis_index("s")
        wid = cid * NS + sid

        for i in range(32):
            for k in range(C // L):
                zbuf[i, pl.ds(k * L, L)] = jnp.zeros((L,), _f32)

        @pl.loop(0, ZR // 32)
        def _(i):
            pltpu.sync_copy(zbuf, acc.at[pl.ds(sid * ZR + i * 32, 32)])

        pltpu.sync_copy(row_hbm.at[wid], row_v)
        pltpu.sync_copy(col_hbm.at[wid], col_v)
        pltpu.sync_copy(w_hbm.at[wid], w_v)
        plsc.subcore_barrier()

        def scale(buf, jj):
            for e in range(CHUNK):
                ws = plsc.load_gather(w_v, [_full(jj * CHUNK + e)])
                for k in range(C // L):
                    sl = pl.ds(k * L, L)
                    buf[e, sl] = buf[e, sl] * ws

        pltpu.async_copy(g_hbm.at[row_v.at[0]], rows0, sem0)

        @pl.loop(0, NCH, step=2)
        def _(j):
            pltpu.async_copy(g_hbm.at[row_v.at[j + 1]], rows1, sem1)
            pltpu.make_async_copy(g_hbm.at[row_v.at[j]], rows0, sem0).wait()
            scale(rows0, j)
            pltpu.sync_copy(rows0, acc.at[col_v.at[j]], add=True)

            @pl.when(j + 2 < NCH)
            def _():
                pltpu.async_copy(g_hbm.at[row_v.at[j + 2]], rows0, sem0)

            pltpu.make_async_copy(g_hbm.at[row_v.at[j + 1]], rows1, sem1).wait()
            scale(rows1, j + 1)
            pltpu.sync_copy(rows1, acc.at[col_v.at[j + 1]], add=True)

        plsc.subcore_barrier()
        pltpu.sync_copy(acc.at[pl.ds(sid * ZR, ZR)],
                        out_hbm.at[pl.ds(cid * N_PAD + sid * ZR, ZR)])

    return msg_kernel


_deg_kernel = _make_deg_kernel()
_msg_kernel16 = _make_msg_kernel(16)
_msg_kernel32 = _make_msg_kernel(32)


_B = 1024
_GRID = N_PAD // _B


def _dis(p0, p1):
    deg = p0 + p1 + 1.0
    return jnp.where(deg > 0, lax.rsqrt(deg), 0.0)


def _node_spec(c):
    return pl.BlockSpec((_B, c), lambda i: (i, 0))


def _full_spec(shape):
    return pl.BlockSpec(shape, lambda i: tuple(0 for _ in shape))


def _tc1_body(x_ref, w1_ref, p0_ref, p1_ref, g1_ref):
    dis = _dis(p0_ref[...], p1_ref[...])
    h = jnp.dot(x_ref[...], w1_ref[...], preferred_element_type=_f32)
    g1_ref[...] = h * dis


def _tc2_body(g1_ref, sa_ref, sb_ref, p0_ref, p1_ref, b1_ref, g2_ref):
    dis = _dis(p0_ref[...], p1_ref[...])
    x2 = jax.nn.relu(dis * (sa_ref[...] + sb_ref[...] + g1_ref[...]) + b1_ref[...])
    g2_ref[...] = dis * x2


def _tc3_body(g2_ref, sa_ref, sb_ref, p0_ref, p1_ref, w2_ref, b2_ref, g3_ref):
    dis = _dis(p0_ref[...], p1_ref[...])
    agg = dis * (sa_ref[...] + sb_ref[...] + g2_ref[...])
    x3 = jax.nn.relu(jnp.dot(agg, w2_ref[...], preferred_element_type=_f32)
                     + b2_ref[...])
    g3_ref[...] = dis * x3


def _tc4_body(g3_ref, sa_ref, sb_ref, p0_ref, p1_ref, w3_ref, b3_ref,
              x_ref, wla_ref, wlb_ref, bl_ref, out_ref):
    dis = _dis(p0_ref[...], p1_ref[...])
    agg = dis * (sa_ref[...] + sb_ref[...] + g3_ref[...])
    x4 = jax.nn.relu(jnp.dot(agg, w3_ref[...], preferred_element_type=_f32)
                     + b3_ref[...])
    out_ref[...] = (jnp.dot(x_ref[...], wla_ref[...], preferred_element_type=_f32)
                    + jnp.dot(x4, wlb_ref[...], preferred_element_type=_f32)
                    + bl_ref[...])


def _tc1(xp, W1, p0, p1):
    return pl.pallas_call(
        _tc1_body,
        grid=(_GRID,),
        in_specs=[_node_spec(128), _full_spec((128, 16)),
                  _node_spec(1), _node_spec(1)],
        out_specs=_node_spec(16),
        out_shape=jax.ShapeDtypeStruct((N_PAD, 16), _f32),
    )(xp, W1, p0, p1)


def _tc2(g1, sa, sb, p0, p1, b1):
    return pl.pallas_call(
        _tc2_body,
        grid=(_GRID,),
        in_specs=[_node_spec(16), _node_spec(16), _node_spec(16),
                  _node_spec(1), _node_spec(1), _full_spec((1, 16))],
        out_specs=_node_spec(16),
        out_shape=jax.ShapeDtypeStruct((N_PAD, 16), _f32),
    )(g1, sa, sb, p0, p1, b1)


def _tc3(g2, sa, sb, p0, p1, W2, b2):
    return pl.pallas_call(
        _tc3_body,
        grid=(_GRID,),
        in_specs=[_node_spec(16), _node_spec(16), _node_spec(16),
                  _node_spec(1), _node_spec(1),
                  _full_spec((16, 32)), _full_spec((1, 32))],
        out_specs=_node_spec(32),
        out_shape=jax.ShapeDtypeStruct((N_PAD, 32), _f32),
    )(g2, sa, sb, p0, p1, W2, b2)


def _tc4(g3, sa, sb, p0, p1, W3, b3, xp, Wla, Wlb, bl):
    return pl.pallas_call(
        _tc4_body,
        grid=(_GRID,),
        in_specs=[_node_spec(32), _node_spec(32), _node_spec(32),
                  _node_spec(1), _node_spec(1),
                  _full_spec((32, 128)), _full_spec((1, 128)),
                  _node_spec(128), _full_spec((128, 128)),
                  _full_spec((128, 128)), _full_spec((1, 128))],
        out_specs=_node_spec(128),
        out_shape=jax.ShapeDtypeStruct((N_PAD, 128), _f32),
    )(g3, sa, sb, p0, p1, W3, b3, xp, Wla, Wlb, bl)


def kernel(x, edge_index, edge_attr, W1, b1, W2, b2, W3, b3, Wl, bl):
    pad_e = E_PAD - E
    rowp = jnp.pad(edge_index[0], (0, pad_e)).reshape(NW, NCH, CHUNK)
    colp = jnp.pad(edge_index[1], (0, pad_e)).reshape(NW, NCH, CHUNK)
    wp = jnp.pad(edge_attr, (0, pad_e)).reshape(NW, EPW)
    xp = jnp.pad(x, ((0, N_PAD - N), (0, 0)))

    deg2 = _deg_kernel(colp, wp)
    p0 = deg2[:N_PAD, None]
    p1 = deg2[N_PAD:, None]

    g1 = _tc1(xp, W1, p0, p1)
    s1 = _msg_kernel16(g1, rowp, colp, wp)
    g2 = _tc2(g1, s1[:N_PAD], s1[N_PAD:], p0, p1, b1.reshape(1, 16))
    s2 = _msg_kernel16(g2, rowp, colp, wp)
    g3 = _tc3(g2, s2[:N_PAD], s2[N_PAD:], p0, p1, W2, b2.reshape(1, 32))
    s3 = _msg_kernel32(g3, rowp, colp, wp)
    out = _tc4(g3, s3[:N_PAD], s3[N_PAD:], p0, p1, W3, b3.reshape(1, 128),
               xp, Wl[:128], Wl[128:], bl.reshape(1, 128))
    return out[:N]
